# Initial kernel scaffold; baseline (speedup 1.0000x reference)
import jax
import jax.numpy as jnp
from jax.experimental import pallas as pl


def kernel(h, edge_index, x1, x2, x1_tar, x2_tar, W_self0, W_neigh0, b0, W_self1, W_neigh1, b1, W_lin1, b_lin1, bn_gamma, bn_beta, W_lin2, b_lin2):
    raise NotImplementedError("write your pallas kernel here")


# speedup vs baseline: 6.9962x; 6.9962x over previous
import jax
import jax.numpy as jnp
from jax import lax
from jax.experimental import pallas as pl
from jax.experimental.pallas import tpu as pltpu
from jax.experimental.pallas import tpu_sc as plsc

_NC = 2
_NS = 16
_HH = 64


def _sc_mesh():
    return plsc.VectorSubcoreMesh(
        core_axis_name="c", subcore_axis_name="s",
        num_cores=_NC, num_subcores=_NS)


def _make_seg_sum(n_pad, e, chunk, with_deg):
    assert e % (_NS * chunk) == 0 and chunk % 8 == 0
    steps = e // (_NS * chunk)
    epw = e // _NS
    rpt = n_pad // _NS

    out_type = [jax.ShapeDtypeStruct((n_pad, _HH), jnp.float32),
                jax.ShapeDtypeStruct((n_pad, _HH), jnp.float32)]
    scratch = [
        pltpu.VMEM((chunk,), jnp.int32),
        pltpu.VMEM((chunk,), jnp.int32),
        pltpu.VMEM((chunk, _HH), jnp.float32),
        pltpu.VMEM_SHARED((n_pad, _HH), jnp.float32),
        pltpu.SemaphoreType.DMA,
    ]
    if with_deg:
        out_type.append(jax.ShapeDtypeStruct((n_pad, 16), jnp.float32))
        scratch += [
            pltpu.VMEM((chunk, 16), jnp.float32),
            pltpu.VMEM_SHARED((n_pad, 16), jnp.float32),
        ]

    def body(*refs):
        if with_deg:
            (h_lo, h_hi, srcs, dsts, z64, z16, ones_h,
             out_lo, out_hi, degp, idx_s, idx_d, rows, acc, sem,
             ones_v, dacc) = refs
        else:
            (h_lo, h_hi, srcs, dsts, z64,
             out_lo, out_hi, idx_s, idx_d, rows, acc, sem) = refs
        c = lax.axis_index("c")
        s = lax.axis_index("s")
        r0 = s * rpt
        pltpu.sync_copy(z64.at[pl.ds(r0, rpt)], acc.at[pl.ds(r0, rpt)])
        if with_deg:
            @pl.when(c == 0)
            def _():
                pltpu.sync_copy(z16.at[pl.ds(r0, rpt)], dacc.at[pl.ds(r0, rpt)])
                pltpu.sync_copy(ones_h, ones_v)
        plsc.subcore_barrier()
        base = s * epw

        def step(i, carry):
            off = pl.multiple_of(base + i * chunk, 8)
            pltpu.sync_copy(srcs.at[pl.ds(off, chunk)], idx_s)
            pltpu.sync_copy(dsts.at[pl.ds(off, chunk)], idx_d)

            @pl.when(c == 0)
            def _():
                pltpu.async_copy(h_lo.at[idx_s], rows, sem).wait()

            @pl.when(c == 1)
            def _():
                pltpu.async_copy(h_hi.at[idx_s], rows, sem).wait()

            pltpu.sync_copy(rows, acc.at[idx_d], add=True)
            if with_deg:
                @pl.when(c == 0)
                def _():
                    pltpu.sync_copy(ones_v, dacc.at[idx_d], add=True)
            return carry

        lax.fori_loop(0, steps, step, 0)
        plsc.subcore_barrier()
        @pl.when(c == 0)
        def _():
            pltpu.sync_copy(acc.at[pl.ds(r0, rpt)], out_lo.at[pl.ds(r0, rpt)])

        @pl.when(c == 1)
        def _():
            pltpu.sync_copy(acc.at[pl.ds(r0, rpt)], out_hi.at[pl.ds(r0, rpt)])
        if with_deg:
            @pl.when(c == 0)
            def _():
                pltpu.sync_copy(dacc.at[pl.ds(r0, rpt)], degp.at[pl.ds(r0, rpt)])

    return pl.kernel(body, out_type=tuple(out_type), mesh=_sc_mesh(),
                     scratch_types=tuple(scratch),
                     compiler_params=pltpu.CompilerParams(
                         use_tc_tiling_on_sc=False))


def _make_row_gather(n_pad, b_tot):
    nw = _NC * _NS
    assert b_tot % nw == 0 and (b_tot // nw) % 8 == 0
    bpw = b_tot // nw

    def body(t_lo, t_hi, idx_hbm, out_lo, out_hi, idx_v, rows_lo,
             rows_hi, sem):
        wid = lax.axis_index("c") * _NS + lax.axis_index("s")
        base = pl.multiple_of(wid * bpw, 8)
        pltpu.sync_copy(idx_hbm.at[pl.ds(base, bpw)], idx_v)
        d1 = pltpu.async_copy(t_lo.at[idx_v], rows_lo, sem)
        d2 = pltpu.async_copy(t_hi.at[idx_v], rows_hi, sem)
        d1.wait()
        d2.wait()
        pltpu.sync_copy(rows_lo, out_lo.at[pl.ds(base, bpw)])
        pltpu.sync_copy(rows_hi, out_hi.at[pl.ds(base, bpw)])

    return pl.kernel(
        body,
        out_type=(jax.ShapeDtypeStruct((b_tot, _HH), jnp.float32),
                  jax.ShapeDtypeStruct((b_tot, _HH), jnp.float32)),
        mesh=_sc_mesh(),
        scratch_types=(
            pltpu.VMEM((bpw,), jnp.int32),
            pltpu.VMEM((bpw, _HH), jnp.float32),
            pltpu.VMEM((bpw, _HH), jnp.float32),
            pltpu.SemaphoreType.DMA,
        ),
        compiler_params=pltpu.CompilerParams(use_tc_tiling_on_sc=False))


def _dense_body(hlo_ref, hhi_ref, plo_ref, phi_ref, d_ref, ws_ref,
                wn_ref, b_ref, olo_ref, ohi_ref):
    h = jnp.concatenate([hlo_ref[...], hhi_ref[...]], axis=1)
    rdeg = 1.0 / jnp.maximum(d_ref[:, 0:1], 1.0)
    hn = jnp.concatenate([plo_ref[...], phi_ref[...]], axis=1) * rdeg
    acc = jnp.dot(h, ws_ref[...], preferred_element_type=jnp.float32)
    acc += jnp.dot(hn, wn_ref[...], preferred_element_type=jnp.float32)
    res = jnp.maximum(acc + b_ref[...], 0.0)
    olo_ref[...] = res[:, :_HH]
    ohi_ref[...] = res[:, _HH:]


def _make_dense(n_pad, blk=1024):
    grid = (n_pad // blk,)
    return pl.pallas_call(
        _dense_body,
        grid=grid,
        in_specs=[
            pl.BlockSpec((blk, _HH), lambda i: (i, 0)),
            pl.BlockSpec((blk, _HH), lambda i: (i, 0)),
            pl.BlockSpec((blk, _HH), lambda i: (i, 0)),
            pl.BlockSpec((blk, _HH), lambda i: (i, 0)),
            pl.BlockSpec((blk, 16), lambda i: (i, 0)),
            pl.BlockSpec((128, 128), lambda i: (0, 0)),
            pl.BlockSpec((128, 128), lambda i: (0, 0)),
            pl.BlockSpec((1, 128), lambda i: (0, 0)),
        ],
        out_specs=[pl.BlockSpec((blk, _HH), lambda i: (i, 0)),
                   pl.BlockSpec((blk, _HH), lambda i: (i, 0))],
        out_shape=[jax.ShapeDtypeStruct((n_pad, _HH), jnp.float32),
                   jax.ShapeDtypeStruct((n_pad, _HH), jnp.float32)],
    )


def _head_body(glo_ref, ghi_ref, w1a, w1b, w1c, b1_ref, gam, bet, w2, b2,
               hp_ref, hs_ref, ht_ref):
    b = hs_ref.shape[0]
    g_all = jnp.concatenate([glo_ref[...], ghi_ref[...]], axis=1)

    def mm(x, w):
        return jnp.dot(x, w[...], preferred_element_type=jnp.float32)

    def one(ha, hb):
        z = mm(ha, w1a) + mm(hb, w1b) + mm(jnp.abs(ha - hb), w1c) + b1_ref[...]
        mu = jnp.mean(z, axis=0, keepdims=True)
        var = jnp.mean((z - mu) ** 2, axis=0, keepdims=True)
        zn = gam[...] * (z - mu) / jnp.sqrt(var + 1e-5) + bet[...]
        return jnp.maximum(zn, 0.0)

    hs = one(g_all[0 * b:1 * b], g_all[1 * b:2 * b])
    ht = one(g_all[2 * b:3 * b], g_all[3 * b:4 * b])
    hs_ref[...] = hs
    ht_ref[...] = ht
    hp_ref[...] = mm(hs, w2) + b2[...]


def _make_head(b):
    full = lambda shape: pl.BlockSpec(shape, lambda: tuple(0 for _ in shape))
    return pl.pallas_call(
        _head_body,
        in_specs=[
            full((4 * b, _HH)), full((4 * b, _HH)),
            full((128, 128)), full((128, 128)), full((128, 128)),
            full((1, 128)), full((1, 128)), full((1, 128)),
            full((128, 128)), full((1, 128)),
        ],
        out_specs=[full((b, 128)), full((b, 128)), full((b, 128))],
        out_shape=[
            jax.ShapeDtypeStruct((b, 128), jnp.float32),
            jax.ShapeDtypeStruct((b, 128), jnp.float32),
            jax.ShapeDtypeStruct((b, 128), jnp.float32),
        ],
    )


def kernel(h, edge_index, x1, x2, x1_tar, x2_tar,
           W_self0, W_neigh0, b0, W_self1, W_neigh1, b1,
           W_lin1, b_lin1, bn_gamma, bn_beta, W_lin2, b_lin2):
    n, d = h.shape
    e = edge_index.shape[1]
    bsz = x1.shape[0]
    hdim = W_self0.shape[1]
    c_out = W_lin2.shape[1]
    n_pad = ((n + 1023) // 1024) * 1024
    chunk = 400
    assert d == 128 and hdim == 128

    src = edge_index[0].astype(jnp.int32)
    dst = edge_index[1].astype(jnp.int32)
    h_pad = jnp.pad(h, ((0, n_pad - n), (0, 0)))
    h_lo = h_pad[:, :_HH]
    h_hi = h_pad[:, _HH:]

    z64 = jnp.zeros((n_pad, _HH), jnp.float32)
    z16 = jnp.zeros((n_pad, 16), jnp.float32)
    ones16 = jnp.ones((chunk, 16), jnp.float32)

    seg0 = _make_seg_sum(n_pad, e, chunk, with_deg=True)
    seg1 = _make_seg_sum(n_pad, e, chunk, with_deg=False)
    dense = _make_dense(n_pad)
    gather = _make_row_gather(n_pad, 4 * bsz)
    head = _make_head(bsz)

    p0_lo, p0_hi, deg = seg0(h_lo, h_hi, src, dst, z64, z16, ones16)
    h1_lo, h1_hi = dense(h_lo, h_hi, p0_lo, p0_hi, deg, W_self0, W_neigh0,
                         b0.reshape(1, 128))
    p1_lo, p1_hi = seg1(h1_lo, h1_hi, src, dst, z64)
    h2_lo, h2_hi = dense(h1_lo, h1_hi, p1_lo, p1_hi, deg, W_self1, W_neigh1,
                         b1.reshape(1, 128))

    idx_all = jnp.concatenate([x1, x2, x1_tar, x2_tar]).astype(jnp.int32)
    g_lo, g_hi = gather(h2_lo, h2_hi, idx_all)

    w1a = W_lin1[0 * hdim:1 * hdim]
    w1b = W_lin1[1 * hdim:2 * hdim]
    w1c = W_lin1[2 * hdim:3 * hdim]
    w2p = jnp.zeros((hdim, 128), jnp.float32).at[:, :c_out].set(W_lin2)
    b2p = jnp.zeros((1, 128), jnp.float32).at[0, :c_out].set(b_lin2)

    hp_pad, hs, ht = head(g_lo, g_hi, w1a, w1b, w1c, b_lin1.reshape(1, 128),
                          bn_gamma.reshape(1, 128), bn_beta.reshape(1, 128),
                          w2p, b2p)
    return hp_pad[:, :c_out], hs, ht


# speedup vs baseline: 10.5377x; 1.5062x over previous
import jax
import jax.numpy as jnp
from jax import lax
from jax.experimental import pallas as pl
from jax.experimental.pallas import tpu as pltpu
from jax.experimental.pallas import tpu_sc as plsc

_NC = 2
_NS = 16
_HH = 64


def _sc_mesh():
    return plsc.VectorSubcoreMesh(
        core_axis_name="c", subcore_axis_name="s",
        num_cores=_NC, num_subcores=_NS)


def _make_seg_sum(n_pad, e, chunk, with_deg):
    assert e % (_NS * chunk) == 0 and chunk % 8 == 0
    steps = e // (_NS * chunk)
    assert steps % 4 == 0 and (steps // 2) % 1 == 0
    half = steps // 2
    epw = e // _NS
    rpt = n_pad // _NS
    nb = 4

    out_type = [jax.ShapeDtypeStruct((n_pad, _HH), jnp.float32),
                jax.ShapeDtypeStruct((n_pad, _HH), jnp.float32)]
    scratch = [
        *[pltpu.VMEM((chunk,), jnp.int32) for _ in range(nb)],
        *[pltpu.VMEM((chunk,), jnp.int32) for _ in range(nb)],
        *[pltpu.VMEM((chunk, _HH), jnp.float32) for _ in range(nb)],
        pltpu.VMEM_SHARED((n_pad, _HH), jnp.float32),
        pltpu.SemaphoreType.DMA,
        pltpu.SemaphoreType.DMA,
    ]
    if with_deg:
        out_type.append(jax.ShapeDtypeStruct((_NC, n_pad, 16), jnp.float32))
        scratch += [
            pltpu.VMEM((chunk, 16), jnp.float32),
            pltpu.VMEM_SHARED((n_pad, 16), jnp.float32),
        ]

    def body(*refs):
        if with_deg:
            (h_lo, h_hi, ei, z64, z16, ones_h, out_lo, out_hi, degp) = refs[:9]
            rest = refs[9:]
        else:
            (h_lo, h_hi, ei, z64, out_lo, out_hi) = refs[:6]
            rest = refs[6:]
        isrc = rest[0:nb]
        idst = rest[nb:2 * nb]
        rows = rest[2 * nb:3 * nb]
        acc, gsem, isem = rest[3 * nb:3 * nb + 3]
        if with_deg:
            ones_v, dacc = rest[3 * nb + 3:]
        c = lax.axis_index("c")
        s = lax.axis_index("s")
        r0 = s * rpt
        pltpu.sync_copy(z64.at[pl.ds(r0, rpt)], acc.at[pl.ds(r0, rpt)])
        if with_deg:
            pltpu.sync_copy(z16.at[pl.ds(r0, rpt)], dacc.at[pl.ds(r0, rpt)])
            pltpu.sync_copy(ones_h, ones_v)
        plsc.subcore_barrier()
        base = s * epw

        def off(g):
            return pl.multiple_of(base + g * chunk, 8)

        def start_gather(b):
            @pl.when(c == 0)
            def _():
                pltpu.async_copy(h_lo.at[isrc[b]], rows[b], gsem)

            @pl.when(c == 1)
            def _():
                pltpu.async_copy(h_hi.at[isrc[b]], rows[b], gsem)

        pltpu.sync_copy(ei.at[0, pl.ds(off(0), chunk)], isrc[0])
        pltpu.sync_copy(ei.at[1, pl.ds(off(0), chunk)], idst[0])
        start_gather(0)
        pltpu.async_copy(ei.at[0, pl.ds(off(1), chunk)], isrc[1], isem)
        pltpu.async_copy(ei.at[1, pl.ds(off(1), chunk)], idst[1], isem)

        def phase(g, b):
            b1 = (b + 1) % nb
            b2 = (b + 2) % nb
            pltpu.make_async_copy(ei.at[0, pl.ds(off(g), chunk)],
                                  isrc[b1], isem).wait()
            pltpu.make_async_copy(ei.at[1, pl.ds(off(g), chunk)],
                                  idst[b1], isem).wait()
            start_gather(b1)
            pltpu.async_copy(ei.at[0, pl.ds(off(g + 2), chunk)], isrc[b2], isem)
            pltpu.async_copy(ei.at[1, pl.ds(off(g + 2), chunk)], idst[b2], isem)
            pltpu.make_async_copy(h_lo.at[isrc[b]], rows[b], gsem).wait()
            pltpu.sync_copy(rows[b], acc.at[idst[b]], add=True)
            if with_deg:
                @pl.when((g < half) == (c == 0))
                def _():
                    pltpu.sync_copy(ones_v, dacc.at[idst[b]], add=True)

        def outer(k, carry):
            g0 = k * nb
            for b in range(nb):
                phase(g0 + b, b)
            return carry

        lax.fori_loop(0, steps // nb, outer, 0)
        pltpu.make_async_copy(h_lo.at[isrc[0]], rows[0], gsem).wait()
        pltpu.make_async_copy(ei.at[0, pl.ds(off(0), chunk)], isrc[1],
                              isem).wait()
        pltpu.make_async_copy(ei.at[1, pl.ds(off(0), chunk)], idst[1],
                              isem).wait()
        plsc.subcore_barrier()

        @pl.when(c == 0)
        def _():
            pltpu.sync_copy(acc.at[pl.ds(r0, rpt)], out_lo.at[pl.ds(r0, rpt)])

        @pl.when(c == 1)
        def _():
            pltpu.sync_copy(acc.at[pl.ds(r0, rpt)], out_hi.at[pl.ds(r0, rpt)])
        if with_deg:
            pltpu.sync_copy(dacc.at[pl.ds(r0, rpt)], degp.at[c, pl.ds(r0, rpt)])

    return pl.kernel(body, out_type=tuple(out_type), mesh=_sc_mesh(),
                     scratch_types=tuple(scratch),
                     compiler_params=pltpu.CompilerParams(
                         use_tc_tiling_on_sc=False))


def _make_row_gather(n_pad, b_tot):
    nw = _NC * _NS
    assert b_tot % nw == 0 and (b_tot // nw) % 8 == 0
    bpw = b_tot // nw

    def body(t_lo, t_hi, idx_hbm, out_lo, out_hi, idx_v, rows_lo,
             rows_hi, sem):
        wid = lax.axis_index("c") * _NS + lax.axis_index("s")
        base = pl.multiple_of(wid * bpw, 8)
        pltpu.sync_copy(idx_hbm.at[pl.ds(base, bpw)], idx_v)
        d1 = pltpu.async_copy(t_lo.at[idx_v], rows_lo, sem)
        d2 = pltpu.async_copy(t_hi.at[idx_v], rows_hi, sem)
        d1.wait()
        d2.wait()
        pltpu.sync_copy(rows_lo, out_lo.at[pl.ds(base, bpw)])
        pltpu.sync_copy(rows_hi, out_hi.at[pl.ds(base, bpw)])

    return pl.kernel(
        body,
        out_type=(jax.ShapeDtypeStruct((b_tot, _HH), jnp.float32),
                  jax.ShapeDtypeStruct((b_tot, _HH), jnp.float32)),
        mesh=_sc_mesh(),
        scratch_types=(
            pltpu.VMEM((bpw,), jnp.int32),
            pltpu.VMEM((bpw, _HH), jnp.float32),
            pltpu.VMEM((bpw, _HH), jnp.float32),
            pltpu.SemaphoreType.DMA,
        ),
        compiler_params=pltpu.CompilerParams(use_tc_tiling_on_sc=False))


def _dense_body(hlo_ref, hhi_ref, plo_ref, phi_ref, d_ref, ws_ref,
                wn_ref, b_ref, olo_ref, ohi_ref):
    h = jnp.concatenate([hlo_ref[...], hhi_ref[...]], axis=1)
    rdeg = 1.0 / jnp.maximum(d_ref[0, :, 0:1] + d_ref[1, :, 0:1], 1.0)
    hn = jnp.concatenate([plo_ref[...], phi_ref[...]], axis=1) * rdeg
    acc = jnp.dot(h, ws_ref[...], preferred_element_type=jnp.float32)
    acc += jnp.dot(hn, wn_ref[...], preferred_element_type=jnp.float32)
    res = jnp.maximum(acc + b_ref[...], 0.0)
    olo_ref[...] = res[:, :_HH]
    ohi_ref[...] = res[:, _HH:]


def _make_dense(n_pad, blk=1024):
    grid = (n_pad // blk,)
    return pl.pallas_call(
        _dense_body,
        grid=grid,
        in_specs=[
            pl.BlockSpec((blk, _HH), lambda i: (i, 0)),
            pl.BlockSpec((blk, _HH), lambda i: (i, 0)),
            pl.BlockSpec((blk, _HH), lambda i: (i, 0)),
            pl.BlockSpec((blk, _HH), lambda i: (i, 0)),
            pl.BlockSpec((_NC, blk, 16), lambda i: (0, i, 0)),
            pl.BlockSpec((128, 128), lambda i: (0, 0)),
            pl.BlockSpec((128, 128), lambda i: (0, 0)),
            pl.BlockSpec((1, 128), lambda i: (0, 0)),
        ],
        out_specs=[pl.BlockSpec((blk, _HH), lambda i: (i, 0)),
                   pl.BlockSpec((blk, _HH), lambda i: (i, 0))],
        out_shape=[jax.ShapeDtypeStruct((n_pad, _HH), jnp.float32),
                   jax.ShapeDtypeStruct((n_pad, _HH), jnp.float32)],
    )


def _head_body(glo_ref, ghi_ref, w1a, w1b, w1c, b1_ref, gam, bet, w2, b2,
               hp_ref, hs_ref, ht_ref):
    b = hs_ref.shape[0]
    g_all = jnp.concatenate([glo_ref[...], ghi_ref[...]], axis=1)

    def mm(x, w):
        return jnp.dot(x, w[...], preferred_element_type=jnp.float32)

    def one(ha, hb):
        z = mm(ha, w1a) + mm(hb, w1b) + mm(jnp.abs(ha - hb), w1c) + b1_ref[...]
        mu = jnp.mean(z, axis=0, keepdims=True)
        var = jnp.mean((z - mu) ** 2, axis=0, keepdims=True)
        zn = gam[...] * (z - mu) / jnp.sqrt(var + 1e-5) + bet[...]
        return jnp.maximum(zn, 0.0)

    hs = one(g_all[0 * b:1 * b], g_all[1 * b:2 * b])
    ht = one(g_all[2 * b:3 * b], g_all[3 * b:4 * b])
    hs_ref[...] = hs
    ht_ref[...] = ht
    hp_ref[...] = mm(hs, w2) + b2[...]


def _make_head(b):
    full = lambda shape: pl.BlockSpec(shape, lambda: tuple(0 for _ in shape))
    return pl.pallas_call(
        _head_body,
        in_specs=[
            full((4 * b, _HH)), full((4 * b, _HH)),
            full((128, 128)), full((128, 128)), full((128, 128)),
            full((1, 128)), full((1, 128)), full((1, 128)),
            full((128, 128)), full((1, 128)),
        ],
        out_specs=[full((b, 128)), full((b, 128)), full((b, 128))],
        out_shape=[
            jax.ShapeDtypeStruct((b, 128), jnp.float32),
            jax.ShapeDtypeStruct((b, 128), jnp.float32),
            jax.ShapeDtypeStruct((b, 128), jnp.float32),
        ],
    )


def kernel(h, edge_index, x1, x2, x1_tar, x2_tar,
           W_self0, W_neigh0, b0, W_self1, W_neigh1, b1,
           W_lin1, b_lin1, bn_gamma, bn_beta, W_lin2, b_lin2):
    n, d = h.shape
    e = edge_index.shape[1]
    bsz = x1.shape[0]
    hdim = W_self0.shape[1]
    c_out = W_lin2.shape[1]
    n_pad = ((n + 1023) // 1024) * 1024
    chunk = 200
    assert d == 128 and hdim == 128

    ei = jnp.pad(edge_index.astype(jnp.int32), ((0, 0), (0, 2 * chunk)))
    h_pad = jnp.pad(h, ((0, n_pad - n), (0, 0)))
    h_lo = h_pad[:, :_HH]
    h_hi = h_pad[:, _HH:]

    z64 = jnp.zeros((n_pad, _HH), jnp.float32)
    z16 = jnp.zeros((n_pad, 16), jnp.float32)
    ones16 = jnp.ones((chunk, 16), jnp.float32)

    seg0 = _make_seg_sum(n_pad, e, chunk, with_deg=True)
    seg1 = _make_seg_sum(n_pad, e, chunk, with_deg=False)
    dense = _make_dense(n_pad)
    gather = _make_row_gather(n_pad, 4 * bsz)
    head = _make_head(bsz)

    p0_lo, p0_hi, deg = seg0(h_lo, h_hi, ei, z64, z16, ones16)
    h1_lo, h1_hi = dense(h_lo, h_hi, p0_lo, p0_hi, deg, W_self0, W_neigh0,
                         b0.reshape(1, 128))
    p1_lo, p1_hi = seg1(h1_lo, h1_hi, ei, z64)
    h2_lo, h2_hi = dense(h1_lo, h1_hi, p1_lo, p1_hi, deg, W_self1, W_neigh1,
                         b1.reshape(1, 128))

    idx_all = jnp.concatenate([x1, x2, x1_tar, x2_tar]).astype(jnp.int32)
    g_lo, g_hi = gather(h2_lo, h2_hi, idx_all)

    w1a = W_lin1[0 * hdim:1 * hdim]
    w1b = W_lin1[1 * hdim:2 * hdim]
    w1c = W_lin1[2 * hdim:3 * hdim]
    w2p = jnp.zeros((hdim, 128), jnp.float32).at[:, :c_out].set(W_lin2)
    b2p = jnp.zeros((1, 128), jnp.float32).at[0, :c_out].set(b_lin2)

    hp_pad, hs, ht = head(g_lo, g_hi, w1a, w1b, w1c, b_lin1.reshape(1, 128),
                          bn_gamma.reshape(1, 128), bn_beta.reshape(1, 128),
                          w2p, b2p)
    return hp_pad[:, :c_out], hs, ht


# speedup vs baseline: 11.6330x; 1.1039x over previous
import jax
import jax.numpy as jnp
from jax import lax
from jax.experimental import pallas as pl
from jax.experimental.pallas import tpu as pltpu
from jax.experimental.pallas import tpu_sc as plsc

_NC = 2
_NS = 16
_HH = 64


def _sc_mesh():
    return plsc.VectorSubcoreMesh(
        core_axis_name="c", subcore_axis_name="s",
        num_cores=_NC, num_subcores=_NS)


def _make_seg_sum(n, n_pad, e, chunk, with_deg):
    assert e % (_NS * chunk) == 0 and chunk % 8 == 0
    steps = e // (_NS * chunk)
    nb = 4
    assert steps % nb == 0
    half = steps // 2
    epw = e // _NS
    rpt = n_pad // _NS

    out_type = [jax.ShapeDtypeStruct((n_pad, _HH), jnp.float32),
                jax.ShapeDtypeStruct((n_pad, _HH), jnp.float32)]
    scratch = [
        *[pltpu.VMEM((2, chunk), jnp.int32) for _ in range(nb)],
        *[pltpu.VMEM((chunk, _HH), jnp.float32) for _ in range(nb)],
        pltpu.VMEM_SHARED((n_pad, _HH), jnp.float32),
        pltpu.SemaphoreType.DMA,
        pltpu.SemaphoreType.DMA,
    ]
    if with_deg:
        out_type.append(jax.ShapeDtypeStruct((_NC, n_pad, 16), jnp.float32))
        scratch += [
            pltpu.VMEM((chunk, 16), jnp.float32),
            pltpu.VMEM_SHARED((n_pad, 16), jnp.float32),
        ]

    def body(*refs):
        if with_deg:
            (h_lo, h_hi, ei, z64, z16, ones_h, out_lo, out_hi, degp) = refs[:9]
            rest = refs[9:]
        else:
            (h_lo, h_hi, ei, z64, out_lo, out_hi) = refs[:6]
            rest = refs[6:]
        idx = rest[0:nb]
        rows = rest[nb:2 * nb]
        acc, gsem, isem = rest[2 * nb:2 * nb + 3]
        if with_deg:
            ones_v, dacc = rest[2 * nb + 3:]
        c = lax.axis_index("c")
        s = lax.axis_index("s")
        r0 = s * rpt
        pltpu.sync_copy(z64.at[pl.ds(r0, rpt)], acc.at[pl.ds(r0, rpt)])
        if with_deg:
            pltpu.sync_copy(z16.at[pl.ds(r0, rpt)], dacc.at[pl.ds(r0, rpt)])
            pltpu.sync_copy(ones_h, ones_v)
        plsc.subcore_barrier()
        base = s * epw

        def off(g):
            gc = jnp.minimum(g, steps - 1) if isinstance(g, jax.Array) \
                else min(g, steps - 1)
            return pl.multiple_of(base + gc * chunk, 8)

        def idx_start(g, slot):
            pltpu.async_copy(ei.at[:, pl.ds(off(g), chunk)], idx[slot], isem)

        def idx_wait(slot):
            pltpu.make_async_copy(ei.at[:, pl.ds(base, chunk)], idx[slot],
                                  isem).wait()

        def gather_start(slot):
            @pl.when(c == 0)
            def _():
                pltpu.async_copy(h_lo.at[idx[slot].at[0]], rows[slot], gsem)

            @pl.when(c == 1)
            def _():
                pltpu.async_copy(h_hi.at[idx[slot].at[0]], rows[slot], gsem)

        def gather_wait(slot):
            pltpu.make_async_copy(h_lo.at[idx[slot].at[0]], rows[slot],
                                  gsem).wait()

        pltpu.sync_copy(ei.at[:, pl.ds(off(0), chunk)], idx[0])
        pltpu.sync_copy(ei.at[:, pl.ds(off(1), chunk)], idx[1])
        gather_start(0)
        gather_start(1)
        idx_start(2, 2)

        def phase(g, b):
            b2 = (b + 2) % nb
            b3 = (b + 3) % nb
            idx_wait(b2)
            gather_start(b2)
            idx_start(g + 3, b3)
            gather_wait(b)
            pltpu.sync_copy(rows[b], acc.at[idx[b].at[1]], add=True)
            if with_deg:
                @pl.when((g < half) == (c == 0))
                def _():
                    pltpu.sync_copy(ones_v, dacc.at[idx[b].at[1]], add=True)

        def outer(k, carry):
            g0 = k * nb
            for b in range(nb):
                phase(g0 + b, b)
            return carry

        lax.fori_loop(0, steps // nb, outer, 0)
        gather_wait(0)
        gather_wait(1)
        idx_wait(2)
        plsc.subcore_barrier()

        @pl.when(c == 0)
        def _():
            pltpu.sync_copy(acc.at[pl.ds(r0, rpt)], out_lo.at[pl.ds(r0, rpt)])

        @pl.when(c == 1)
        def _():
            pltpu.sync_copy(acc.at[pl.ds(r0, rpt)], out_hi.at[pl.ds(r0, rpt)])
        if with_deg:
            pltpu.sync_copy(dacc.at[pl.ds(r0, rpt)], degp.at[c, pl.ds(r0, rpt)])

    return pl.kernel(body, out_type=tuple(out_type), mesh=_sc_mesh(),
                     scratch_types=tuple(scratch),
                     compiler_params=pltpu.CompilerParams(
                         use_tc_tiling_on_sc=False))


def _make_row_gather(b_tot):
    nw = _NC * _NS
    assert b_tot % nw == 0 and (b_tot // nw) % 8 == 0
    bpw = b_tot // nw

    def body(t_lo, t_hi, idx_hbm, out_lo, out_hi, idx_v, rows_lo,
             rows_hi, sem):
        wid = lax.axis_index("c") * _NS + lax.axis_index("s")
        base = pl.multiple_of(wid * bpw, 8)
        pltpu.sync_copy(idx_hbm.at[pl.ds(base, bpw)], idx_v)
        d1 = pltpu.async_copy(t_lo.at[idx_v], rows_lo, sem)
        d2 = pltpu.async_copy(t_hi.at[idx_v], rows_hi, sem)
        d1.wait()
        d2.wait()
        pltpu.sync_copy(rows_lo, out_lo.at[pl.ds(base, bpw)])
        pltpu.sync_copy(rows_hi, out_hi.at[pl.ds(base, bpw)])

    return pl.kernel(
        body,
        out_type=(jax.ShapeDtypeStruct((b_tot, _HH), jnp.float32),
                  jax.ShapeDtypeStruct((b_tot, _HH), jnp.float32)),
        mesh=_sc_mesh(),
        scratch_types=(
            pltpu.VMEM((bpw,), jnp.int32),
            pltpu.VMEM((bpw, _HH), jnp.float32),
            pltpu.VMEM((bpw, _HH), jnp.float32),
            pltpu.SemaphoreType.DMA,
        ),
        compiler_params=pltpu.CompilerParams(use_tc_tiling_on_sc=False))


def _dense_body(hlo_ref, hhi_ref, plo_ref, phi_ref, d_ref, ws_ref,
                wn_ref, b_ref, olo_ref, ohi_ref):
    h = jnp.concatenate([hlo_ref[...], hhi_ref[...]], axis=1)
    rdeg = 1.0 / jnp.maximum(d_ref[0, :, 0:1] + d_ref[1, :, 0:1], 1.0)
    hn = jnp.concatenate([plo_ref[...], phi_ref[...]], axis=1) * rdeg
    acc = jnp.dot(h, ws_ref[...], preferred_element_type=jnp.float32)
    acc += jnp.dot(hn, wn_ref[...], preferred_element_type=jnp.float32)
    res = jnp.maximum(acc + b_ref[...], 0.0)
    olo_ref[...] = res[:, :_HH]
    ohi_ref[...] = res[:, _HH:]


def _make_dense(n, n_pad, blk=1024):
    grid = (n_pad // blk,)
    return pl.pallas_call(
        _dense_body,
        grid=grid,
        in_specs=[
            pl.BlockSpec((blk, _HH), lambda i: (i, 0)),
            pl.BlockSpec((blk, _HH), lambda i: (i, 0)),
            pl.BlockSpec((blk, _HH), lambda i: (i, 0)),
            pl.BlockSpec((blk, _HH), lambda i: (i, 0)),
            pl.BlockSpec((_NC, blk, 16), lambda i: (0, i, 0)),
            pl.BlockSpec((128, 128), lambda i: (0, 0)),
            pl.BlockSpec((128, 128), lambda i: (0, 0)),
            pl.BlockSpec((1, 128), lambda i: (0, 0)),
        ],
        out_specs=[pl.BlockSpec((blk, _HH), lambda i: (i, 0)),
                   pl.BlockSpec((blk, _HH), lambda i: (i, 0))],
        out_shape=[jax.ShapeDtypeStruct((n, _HH), jnp.float32),
                   jax.ShapeDtypeStruct((n, _HH), jnp.float32)],
    )


def _head_body(glo_ref, ghi_ref, w1a, w1b, w1c, b1_ref, gam, bet, w2, b2,
               hp_ref, hs_ref, ht_ref):
    b = hs_ref.shape[0]
    g_all = jnp.concatenate([glo_ref[...], ghi_ref[...]], axis=1)

    def mm(x, w):
        return jnp.dot(x, w[...], preferred_element_type=jnp.float32)

    def one(ha, hb):
        z = mm(ha, w1a) + mm(hb, w1b) + mm(jnp.abs(ha - hb), w1c) + b1_ref[...]
        mu = jnp.mean(z, axis=0, keepdims=True)
        var = jnp.mean((z - mu) ** 2, axis=0, keepdims=True)
        zn = gam[...] * (z - mu) / jnp.sqrt(var + 1e-5) + bet[...]
        return jnp.maximum(zn, 0.0)

    hs = one(g_all[0 * b:1 * b], g_all[1 * b:2 * b])
    ht = one(g_all[2 * b:3 * b], g_all[3 * b:4 * b])
    hs_ref[...] = hs
    ht_ref[...] = ht
    hp_ref[...] = mm(hs, w2) + b2[...]


def _make_head(b):
    full = lambda shape: pl.BlockSpec(shape, lambda: tuple(0 for _ in shape))
    return pl.pallas_call(
        _head_body,
        in_specs=[
            full((4 * b, _HH)), full((4 * b, _HH)),
            full((128, 128)), full((128, 128)), full((128, 128)),
            full((1, 128)), full((1, 128)), full((1, 128)),
            full((128, 128)), full((1, 128)),
        ],
        out_specs=[full((b, 128)), full((b, 128)), full((b, 128))],
        out_shape=[
            jax.ShapeDtypeStruct((b, 128), jnp.float32),
            jax.ShapeDtypeStruct((b, 128), jnp.float32),
            jax.ShapeDtypeStruct((b, 128), jnp.float32),
        ],
    )


def kernel(h, edge_index, x1, x2, x1_tar, x2_tar,
           W_self0, W_neigh0, b0, W_self1, W_neigh1, b1,
           W_lin1, b_lin1, bn_gamma, bn_beta, W_lin2, b_lin2):
    n, d = h.shape
    e = edge_index.shape[1]
    bsz = x1.shape[0]
    hdim = W_self0.shape[1]
    c_out = W_lin2.shape[1]
    n_pad = ((n + 1023) // 1024) * 1024
    chunk = 200
    assert d == 128 and hdim == 128

    ei = edge_index.astype(jnp.int32)
    h_lo = h[:, :_HH]
    h_hi = h[:, _HH:]

    z64 = jnp.zeros((n_pad, _HH), jnp.float32)
    z16 = jnp.zeros((n_pad, 16), jnp.float32)
    ones16 = jnp.ones((chunk, 16), jnp.float32)

    seg0 = _make_seg_sum(n, n_pad, e, chunk, with_deg=True)
    seg1 = _make_seg_sum(n, n_pad, e, chunk, with_deg=False)
    dense = _make_dense(n, n_pad)
    gather = _make_row_gather(4 * bsz)
    head = _make_head(bsz)

    p0_lo, p0_hi, deg = seg0(h_lo, h_hi, ei, z64, z16, ones16)
    h1_lo, h1_hi = dense(h_lo, h_hi, p0_lo, p0_hi, deg, W_self0, W_neigh0,
                         b0.reshape(1, 128))
    p1_lo, p1_hi = seg1(h1_lo, h1_hi, ei, z64)
    h2_lo, h2_hi = dense(h1_lo, h1_hi, p1_lo, p1_hi, deg, W_self1, W_neigh1,
                         b1.reshape(1, 128))

    idx_all = jnp.concatenate([x1, x2, x1_tar, x2_tar]).astype(jnp.int32)
    g_lo, g_hi = gather(h2_lo, h2_hi, idx_all)

    w1a = W_lin1[0 * hdim:1 * hdim]
    w1b = W_lin1[1 * hdim:2 * hdim]
    w1c = W_lin1[2 * hdim:3 * hdim]
    w2p = jnp.zeros((hdim, 128), jnp.float32).at[:, :c_out].set(W_lin2)
    b2p = jnp.zeros((1, 128), jnp.float32).at[0, :c_out].set(b_lin2)

    hp_pad, hs, ht = head(g_lo, g_hi, w1a, w1b, w1c, b_lin1.reshape(1, 128),
                          bn_gamma.reshape(1, 128), bn_beta.reshape(1, 128),
                          w2p, b2p)
    return hp_pad[:, :c_out], hs, ht


# speedup vs baseline: 12.5870x; 1.0820x over previous
import jax
import jax.numpy as jnp
from jax import lax
from jax.experimental import pallas as pl
from jax.experimental.pallas import tpu as pltpu
from jax.experimental.pallas import tpu_sc as plsc

_NC = 2
_NS = 16
_HH = 64


def _sc_mesh():
    return plsc.VectorSubcoreMesh(
        core_axis_name="c", subcore_axis_name="s",
        num_cores=_NC, num_subcores=_NS)


def _chunks(total, step):
    out, o = [], 0
    while o < total:
        sz = min(step, total - o)
        out.append((o, sz))
        o += sz
    assert all(sz % 8 == 0 for _, sz in out)
    return out


def _make_seg(n, n_pad, e, chunk, b4, with_deg):
    assert e % (_NS * chunk) == 0 and chunk % 8 == 0
    steps = e // (_NS * chunk)
    nb = 4
    assert steps % nb == 0
    half = steps // 2
    epw = e // _NS
    rpt = n_pad // _NS

    if with_deg:
        out_type = [jax.ShapeDtypeStruct((n_pad, 128), jnp.float32),
                    jax.ShapeDtypeStruct((n_pad, 16), jnp.float32),
                    jax.ShapeDtypeStruct((n_pad, 16), jnp.float32)]
    else:
        out_type = [jax.ShapeDtypeStruct((b4, 128), jnp.float32),
                    jax.ShapeDtypeStruct((b4, 128), jnp.float32),
                    jax.ShapeDtypeStruct((b4, 16), jnp.float32),
                    jax.ShapeDtypeStruct((b4, 16), jnp.float32)]
    scratch = [
        *[pltpu.VMEM((2, chunk), jnp.int32) for _ in range(nb)],
        *[pltpu.VMEM((chunk, _HH), jnp.float32) for _ in range(nb)],
        pltpu.VMEM_SHARED((n_pad, _HH), jnp.float32),
        pltpu.SemaphoreType.DMA,
        pltpu.SemaphoreType.DMA,
    ]
    if with_deg:
        scratch += [
            pltpu.VMEM((chunk, 16), jnp.float32),
            pltpu.VMEM_SHARED((n_pad, 16), jnp.float32),
        ]
    else:
        scratch += [
            pltpu.VMEM((chunk, 16), jnp.float32),
            pltpu.VMEM((chunk, 16), jnp.float32),
        ]

    def body(*refs):
        if with_deg:
            (h_lo, h_hi, ei, z64, z16, ones_h, out, d0, d1) = refs[:9]
            rest = refs[9:]
        else:
            (h_lo, h_hi, ei, z64, idx_hbm, d0, d1,
             g_h, g_p, g_d0, g_d1) = refs[:11]
            rest = refs[11:]
        idx = rest[0:nb]
        rows = rest[nb:2 * nb]
        acc, gsem, isem = rest[2 * nb:2 * nb + 3]
        if with_deg:
            ones_v, dacc = rest[2 * nb + 3:]
        else:
            db0, db1 = rest[2 * nb + 3:]
        c = lax.axis_index("c")
        s = lax.axis_index("s")
        r0 = s * rpt
        pltpu.sync_copy(z64.at[pl.ds(r0, rpt)], acc.at[pl.ds(r0, rpt)])
        if with_deg:
            pltpu.sync_copy(z16.at[pl.ds(r0, rpt)], dacc.at[pl.ds(r0, rpt)])
            pltpu.sync_copy(ones_h, ones_v)
        plsc.subcore_barrier()
        base = s * epw

        def off(g):
            gc = jnp.minimum(g, steps - 1) if isinstance(g, jax.Array) \
                else min(g, steps - 1)
            return pl.multiple_of(base + gc * chunk, 8)

        def idx_start(g, slot):
            pltpu.async_copy(ei.at[:, pl.ds(off(g), chunk)], idx[slot], isem)

        def idx_wait(slot):
            pltpu.make_async_copy(ei.at[:, pl.ds(base, chunk)], idx[slot],
                                  isem).wait()

        def gather_start(slot):
            @pl.when(c == 0)
            def _():
                pltpu.async_copy(h_lo.at[idx[slot].at[0]], rows[slot], gsem)

            @pl.when(c == 1)
            def _():
                pltpu.async_copy(h_hi.at[idx[slot].at[0]], rows[slot], gsem)

        def gather_wait(slot):
            pltpu.make_async_copy(h_lo.at[idx[slot].at[0]], rows[slot],
                                  gsem).wait()

        pltpu.sync_copy(ei.at[:, pl.ds(off(0), chunk)], idx[0])
        pltpu.sync_copy(ei.at[:, pl.ds(off(1), chunk)], idx[1])
        gather_start(0)
        gather_start(1)
        idx_start(2, 2)

        def phase(g, b):
            b2 = (b + 2) % nb
            b3 = (b + 3) % nb
            idx_wait(b2)
            gather_start(b2)
            idx_start(g + 3, b3)
            gather_wait(b)
            pltpu.sync_copy(rows[b], acc.at[idx[b].at[1]], add=True)
            if with_deg:
                @pl.when((g < half) == (c == 0))
                def _():
                    pltpu.sync_copy(ones_v, dacc.at[idx[b].at[1]], add=True)

        def outer(k, carry):
            g0 = k * nb
            for b in range(nb):
                phase(g0 + b, b)
            return carry

        lax.fori_loop(0, steps // nb, outer, 0)
        gather_wait(0)
        gather_wait(1)
        idx_wait(2)
        plsc.subcore_barrier()

        if with_deg:
            pltpu.sync_copy(acc.at[pl.ds(r0, rpt)],
                            out.at[pl.ds(r0, rpt), pl.ds(c * _HH, _HH)])

            @pl.when(c == 0)
            def _():
                pltpu.sync_copy(dacc.at[pl.ds(r0, rpt)], d0.at[pl.ds(r0, rpt)])

            @pl.when(c == 1)
            def _():
                pltpu.sync_copy(dacc.at[pl.ds(r0, rpt)], d1.at[pl.ds(r0, rpt)])
            return

        wid = c * _NS + s
        gb = wid * (b4 // (_NC * _NS))
        for o, sz in _chunks(b4 // (_NC * _NS), chunk):
            ib = idx[0].at[0, pl.ds(0, sz)]
            pltpu.sync_copy(idx_hbm.at[pl.ds(gb + o, sz)],
                            idx[0].at[0, pl.ds(0, sz)])
            a1 = pltpu.async_copy(h_lo.at[ib], rows[0].at[pl.ds(0, sz)], gsem)
            a2 = pltpu.async_copy(h_hi.at[ib], rows[1].at[pl.ds(0, sz)], gsem)
            a3 = pltpu.async_copy(d0.at[ib], db0.at[pl.ds(0, sz)], gsem)
            a4 = pltpu.async_copy(d1.at[ib], db1.at[pl.ds(0, sz)], gsem)
            a1.wait()
            a2.wait()
            a3.wait()
            a4.wait()
            pltpu.sync_copy(rows[0].at[pl.ds(0, sz)],
                            g_h.at[pl.ds(gb + o, sz), pl.ds(0, _HH)])
            pltpu.sync_copy(rows[1].at[pl.ds(0, sz)],
                            g_h.at[pl.ds(gb + o, sz), pl.ds(_HH, _HH)])
            pltpu.sync_copy(db0.at[pl.ds(0, sz)], g_d0.at[pl.ds(gb + o, sz)])
            pltpu.sync_copy(db1.at[pl.ds(0, sz)], g_d1.at[pl.ds(gb + o, sz)])
        pb = s * (b4 // _NS)
        for o, sz in _chunks(b4 // _NS, chunk):
            ibp = idx[1].at[0, pl.ds(0, sz)]
            pltpu.sync_copy(idx_hbm.at[pl.ds(pb + o, sz)],
                            idx[1].at[0, pl.ds(0, sz)])
            pltpu.async_copy(acc.at[ibp], rows[2].at[pl.ds(0, sz)],
                             gsem).wait()
            pltpu.sync_copy(rows[2].at[pl.ds(0, sz)],
                            g_p.at[pl.ds(pb + o, sz), pl.ds(c * _HH, _HH)])

    return pl.kernel(body, out_type=tuple(out_type), mesh=_sc_mesh(),
                     scratch_types=tuple(scratch),
                     compiler_params=pltpu.CompilerParams(
                         use_tc_tiling_on_sc=False))


def _dense_body(hlo_ref, hhi_ref, p_ref, d0_ref, d1_ref, ws_ref,
                wn_ref, b_ref, olo_ref, ohi_ref):
    h = jnp.concatenate([hlo_ref[...], hhi_ref[...]], axis=1)
    rdeg = 1.0 / jnp.maximum(d0_ref[:, 0:1] + d1_ref[:, 0:1], 1.0)
    hn = p_ref[...] * rdeg
    acc = jnp.dot(h, ws_ref[...], preferred_element_type=jnp.float32)
    acc += jnp.dot(hn, wn_ref[...], preferred_element_type=jnp.float32)
    res = jnp.maximum(acc + b_ref[...], 0.0)
    olo_ref[...] = res[:, :_HH]
    ohi_ref[...] = res[:, _HH:]


def _make_dense(n, n_pad, blk=1024):
    grid = (n_pad // blk,)
    return pl.pallas_call(
        _dense_body,
        grid=grid,
        in_specs=[
            pl.BlockSpec((blk, _HH), lambda i: (i, 0)),
            pl.BlockSpec((blk, _HH), lambda i: (i, 0)),
            pl.BlockSpec((blk, 128), lambda i: (i, 0)),
            pl.BlockSpec((blk, 16), lambda i: (i, 0)),
            pl.BlockSpec((blk, 16), lambda i: (i, 0)),
            pl.BlockSpec((128, 128), lambda i: (0, 0)),
            pl.BlockSpec((128, 128), lambda i: (0, 0)),
            pl.BlockSpec((1, 128), lambda i: (0, 0)),
        ],
        out_specs=[pl.BlockSpec((blk, _HH), lambda i: (i, 0)),
                   pl.BlockSpec((blk, _HH), lambda i: (i, 0))],
        out_shape=[jax.ShapeDtypeStruct((n, _HH), jnp.float32),
                   jax.ShapeDtypeStruct((n, _HH), jnp.float32)],
    )


def _head_body(gh_ref, gp_ref, gd0_ref, gd1_ref, ws1, wn1, b1_ref,
               w1a, w1b, w1c, bl1, gam, bet, w2, b2,
               hp_ref, hs_ref, ht_ref):
    b = hs_ref.shape[0]

    def mm(x, w):
        return jnp.dot(x, w[...], preferred_element_type=jnp.float32)

    rdeg = 1.0 / jnp.maximum(gd0_ref[:, 0:1] + gd1_ref[:, 0:1], 1.0)
    h2 = mm(gh_ref[...], ws1) + mm(gp_ref[...] * rdeg, wn1) + b1_ref[...]
    h2 = jnp.maximum(h2, 0.0)
    ha, hb = h2[:b], h2[b:]
    z = mm(ha, w1a) + mm(hb, w1b) + mm(jnp.abs(ha - hb), w1c) + bl1[...]
    mu = jnp.mean(z, axis=0, keepdims=True)
    var = jnp.mean((z - mu) ** 2, axis=0, keepdims=True)
    zn = gam[...] * (z - mu) / jnp.sqrt(var + 1e-5) + bet[...]
    res = jnp.maximum(zn, 0.0)
    i = pl.program_id(0)

    @pl.when(i == 0)
    def _():
        hs_ref[...] = res
        hp_ref[...] = mm(res, w2) + b2[...]

    @pl.when(i == 1)
    def _():
        ht_ref[...] = res


def _make_head(b):
    w = lambda shape: pl.BlockSpec(shape, lambda i: tuple(0 for _ in shape))
    return pl.pallas_call(
        _head_body,
        grid=(2,),
        in_specs=[
            pl.BlockSpec((2 * b, 128), lambda i: (i, 0)),
            pl.BlockSpec((2 * b, 128), lambda i: (i, 0)),
            pl.BlockSpec((2 * b, 16), lambda i: (i, 0)),
            pl.BlockSpec((2 * b, 16), lambda i: (i, 0)),
            w((128, 128)), w((128, 128)), w((1, 128)),
            w((128, 128)), w((128, 128)), w((128, 128)),
            w((1, 128)), w((1, 128)), w((1, 128)),
            w((128, 128)), w((1, 128)),
        ],
        out_specs=[w((b, 128)), w((b, 128)), w((b, 128))],
        out_shape=[
            jax.ShapeDtypeStruct((b, 128), jnp.float32),
            jax.ShapeDtypeStruct((b, 128), jnp.float32),
            jax.ShapeDtypeStruct((b, 128), jnp.float32),
        ],
    )


def kernel(h, edge_index, x1, x2, x1_tar, x2_tar,
           W_self0, W_neigh0, b0, W_self1, W_neigh1, b1,
           W_lin1, b_lin1, bn_gamma, bn_beta, W_lin2, b_lin2):
    n, d = h.shape
    e = edge_index.shape[1]
    bsz = x1.shape[0]
    hdim = W_self0.shape[1]
    c_out = W_lin2.shape[1]
    n_pad = ((n + 1023) // 1024) * 1024
    chunk = 200
    b4 = 4 * bsz
    assert d == 128 and hdim == 128

    ei = edge_index.astype(jnp.int32)
    h_lo = h[:, :_HH]
    h_hi = h[:, _HH:]

    z64 = jnp.zeros((n_pad, _HH), jnp.float32)
    z16 = jnp.zeros((n_pad, 16), jnp.float32)
    ones16 = jnp.ones((chunk, 16), jnp.float32)

    seg0 = _make_seg(n, n_pad, e, chunk, b4, with_deg=True)
    seg1 = _make_seg(n, n_pad, e, chunk, b4, with_deg=False)
    dense = _make_dense(n, n_pad)
    head = _make_head(bsz)

    idx_all = jnp.concatenate([x1, x2, x1_tar, x2_tar]).astype(jnp.int32)

    p0, d0, d1 = seg0(h_lo, h_hi, ei, z64, z16, ones16)
    h1_lo, h1_hi = dense(h_lo, h_hi, p0, d0, d1, W_self0, W_neigh0,
                         b0.reshape(1, 128))
    g_h, g_p, g_d0, g_d1 = seg1(h1_lo, h1_hi, ei, z64, idx_all, d0, d1)

    w1a = W_lin1[0 * hdim:1 * hdim]
    w1b = W_lin1[1 * hdim:2 * hdim]
    w1c = W_lin1[2 * hdim:3 * hdim]
    w2p = jnp.zeros((hdim, 128), jnp.float32).at[:, :c_out].set(W_lin2)
    b2p = jnp.zeros((1, 128), jnp.float32).at[0, :c_out].set(b_lin2)

    hp_pad, hs, ht = head(g_h, g_p, g_d0, g_d1, W_self1, W_neigh1,
                          b1.reshape(1, 128), w1a, w1b, w1c,
                          b_lin1.reshape(1, 128), bn_gamma.reshape(1, 128),
                          bn_beta.reshape(1, 128), w2p, b2p)
    return hp_pad[:, :c_out], hs, ht


# speedup vs baseline: 12.8129x; 1.0179x over previous
import jax
import jax.numpy as jnp
from jax import lax
from jax.experimental import pallas as pl
from jax.experimental.pallas import tpu as pltpu
from jax.experimental.pallas import tpu_sc as plsc

_NC = 2
_NS = 16
_HH = 64


def _sc_mesh():
    return plsc.VectorSubcoreMesh(
        core_axis_name="c", subcore_axis_name="s",
        num_cores=_NC, num_subcores=_NS)


def _chunks(total, step):
    out, o = [], 0
    while o < total:
        sz = min(step, total - o)
        out.append((o, sz))
        o += sz
    assert all(sz % 8 == 0 for _, sz in out)
    return out


def _make_seg(n, n_pad, e, chunk, b4, with_deg):
    assert e % (_NS * chunk) == 0 and chunk % 8 == 0
    steps = e // (_NS * chunk)
    nb = 4
    assert steps % nb == 0
    half = steps // 2
    epw = e // _NS
    rpt = n_pad // _NS

    if with_deg:
        out_type = [jax.ShapeDtypeStruct((n_pad, 128), jnp.float32),
                    jax.ShapeDtypeStruct((n_pad, 16), jnp.float32),
                    jax.ShapeDtypeStruct((n_pad, 16), jnp.float32)]
    else:
        out_type = [jax.ShapeDtypeStruct((b4, 128), jnp.float32),
                    jax.ShapeDtypeStruct((b4, 128), jnp.float32),
                    jax.ShapeDtypeStruct((b4, 16), jnp.float32)]
    scratch = [
        *[pltpu.VMEM((2, chunk), jnp.int32) for _ in range(nb)],
        *[pltpu.VMEM((chunk, _HH), jnp.float32) for _ in range(nb)],
        pltpu.VMEM_SHARED((n_pad, _HH), jnp.float32),
        pltpu.SemaphoreType.DMA,
        pltpu.SemaphoreType.DMA,
    ]
    if with_deg:
        scratch += [
            pltpu.VMEM((chunk, 16), jnp.float32),
            pltpu.VMEM_SHARED((n_pad, 16), jnp.float32),
        ]
    else:
        scratch += [
            pltpu.VMEM((chunk, 16), jnp.float32),
            pltpu.VMEM((chunk, 16), jnp.float32),
        ]

    def body(*refs):
        if with_deg:
            (h_lo, h_hi, ei, z64, z16, ones_h, out, d0, d1) = refs[:9]
            rest = refs[9:]
        else:
            (h_lo, h_hi, ei, z64, idx_hbm, rd,
             g_h, g_p, g_r) = refs[:9]
            rest = refs[9:]
        idx = rest[0:nb]
        rows = rest[nb:2 * nb]
        acc, gsem, isem = rest[2 * nb:2 * nb + 3]
        if with_deg:
            ones_v, dacc = rest[2 * nb + 3:]
        else:
            db0, db1 = rest[2 * nb + 3:]
        c = lax.axis_index("c")
        s = lax.axis_index("s")
        r0 = s * rpt
        pltpu.sync_copy(z64.at[pl.ds(r0, rpt)], acc.at[pl.ds(r0, rpt)])
        if with_deg:
            pltpu.sync_copy(z16.at[pl.ds(r0, rpt)], dacc.at[pl.ds(r0, rpt)])
            pltpu.sync_copy(ones_h, ones_v)
        plsc.subcore_barrier()
        base = s * epw

        def off(g):
            gc = jnp.minimum(g, steps - 1) if isinstance(g, jax.Array) \
                else min(g, steps - 1)
            return pl.multiple_of(base + gc * chunk, 8)

        def idx_start(g, slot):
            pltpu.async_copy(ei.at[:, pl.ds(off(g), chunk)], idx[slot], isem)

        def idx_wait(slot):
            pltpu.make_async_copy(ei.at[:, pl.ds(base, chunk)], idx[slot],
                                  isem).wait()

        def gather_start(slot):
            @pl.when(c == 0)
            def _():
                pltpu.async_copy(h_lo.at[idx[slot].at[0]], rows[slot], gsem)

            @pl.when(c == 1)
            def _():
                pltpu.async_copy(h_hi.at[idx[slot].at[0]], rows[slot], gsem)

        def gather_wait(slot):
            pltpu.make_async_copy(h_lo.at[idx[slot].at[0]], rows[slot],
                                  gsem).wait()

        pltpu.sync_copy(ei.at[:, pl.ds(off(0), chunk)], idx[0])
        pltpu.sync_copy(ei.at[:, pl.ds(off(1), chunk)], idx[1])
        gather_start(0)
        gather_start(1)
        idx_start(2, 2)

        def phase(g, b):
            b2 = (b + 2) % nb
            b3 = (b + 3) % nb
            idx_wait(b2)
            gather_start(b2)
            idx_start(g + 3, b3)
            gather_wait(b)
            pltpu.sync_copy(rows[b], acc.at[idx[b].at[1]], add=True)
            if with_deg:
                @pl.when((g < half) == (c == 0))
                def _():
                    pltpu.sync_copy(ones_v, dacc.at[idx[b].at[1]], add=True)

        def outer(k, carry):
            g0 = k * nb
            for b in range(nb):
                phase(g0 + b, b)
            return carry

        lax.fori_loop(0, steps // nb, outer, 0)
        gather_wait(0)
        gather_wait(1)
        idx_wait(2)
        plsc.subcore_barrier()

        if with_deg:
            pltpu.sync_copy(acc.at[pl.ds(r0, rpt)],
                            out.at[pl.ds(r0, rpt), pl.ds(c * _HH, _HH)])

            @pl.when(c == 0)
            def _():
                pltpu.sync_copy(dacc.at[pl.ds(r0, rpt)], d0.at[pl.ds(r0, rpt)])

            @pl.when(c == 1)
            def _():
                pltpu.sync_copy(dacc.at[pl.ds(r0, rpt)], d1.at[pl.ds(r0, rpt)])
            return

        wid = c * _NS + s
        bpw = b4 // (_NC * _NS)
        gb = wid * bpw
        achunks = _chunks(bpw, chunk)
        asets = [(rows[0], rows[1], db0, idx[0]),
                 (rows[2], rows[3], db1, idx[2])]

        def a_fire(k):
            o, sz = achunks[k]
            r1, r2, dbuf, ix = asets[k % 2]
            pltpu.sync_copy(idx_hbm.at[pl.ds(gb + o, sz)],
                            ix.at[0, pl.ds(0, sz)])
            ib = ix.at[0, pl.ds(0, sz)]
            pltpu.async_copy(h_lo.at[ib], r1.at[pl.ds(0, sz)], gsem)
            pltpu.async_copy(h_hi.at[ib], r2.at[pl.ds(0, sz)], gsem)
            pltpu.async_copy(rd.at[ib], dbuf.at[pl.ds(0, sz)], gsem)

        def a_drain(k):
            o, sz = achunks[k]
            r1, r2, dbuf, ix = asets[k % 2]
            ib = ix.at[0, pl.ds(0, sz)]
            pltpu.make_async_copy(h_lo.at[ib], r1.at[pl.ds(0, sz)],
                                  gsem).wait()
            pltpu.make_async_copy(h_hi.at[ib], r2.at[pl.ds(0, sz)],
                                  gsem).wait()
            pltpu.make_async_copy(rd.at[ib], dbuf.at[pl.ds(0, sz)],
                                  gsem).wait()
            pltpu.sync_copy(r1.at[pl.ds(0, sz)],
                            g_h.at[pl.ds(gb + o, sz), pl.ds(0, _HH)])
            pltpu.sync_copy(r2.at[pl.ds(0, sz)],
                            g_h.at[pl.ds(gb + o, sz), pl.ds(_HH, _HH)])
            pltpu.sync_copy(dbuf.at[pl.ds(0, sz)], g_r.at[pl.ds(gb + o, sz)])

        a_fire(0)
        for k in range(len(achunks)):
            if k + 1 < len(achunks):
                a_fire(k + 1)
            a_drain(k)

        ppw = b4 // _NS
        pb = s * ppw
        bchunks = _chunks(ppw, chunk)
        bsets = [(rows[0], idx[1]), (rows[1], idx[3])]

        def b_fire(k):
            o, sz = bchunks[k]
            rb, ix = bsets[k % 2]
            pltpu.sync_copy(idx_hbm.at[pl.ds(pb + o, sz)],
                            ix.at[0, pl.ds(0, sz)])
            pltpu.async_copy(acc.at[ix.at[0, pl.ds(0, sz)]],
                             rb.at[pl.ds(0, sz)], gsem)

        def b_drain(k):
            o, sz = bchunks[k]
            rb, ix = bsets[k % 2]
            pltpu.make_async_copy(acc.at[ix.at[0, pl.ds(0, sz)]],
                                  rb.at[pl.ds(0, sz)], gsem).wait()
            pltpu.sync_copy(rb.at[pl.ds(0, sz)],
                            g_p.at[pl.ds(pb + o, sz), pl.ds(c * _HH, _HH)])

        b_fire(0)
        for k in range(len(bchunks)):
            if k + 1 < len(bchunks):
                b_fire(k + 1)
            b_drain(k)

    return pl.kernel(body, out_type=tuple(out_type), mesh=_sc_mesh(),
                     scratch_types=tuple(scratch),
                     compiler_params=pltpu.CompilerParams(
                         use_tc_tiling_on_sc=False))


def _dense_body(hlo_ref, hhi_ref, p_ref, d0_ref, d1_ref, ws_ref,
                wn_ref, b_ref, olo_ref, ohi_ref, ord_ref):
    h = jnp.concatenate([hlo_ref[...], hhi_ref[...]], axis=1)
    rdeg = 1.0 / jnp.maximum(d0_ref[:, 0:1] + d1_ref[:, 0:1], 1.0)
    hn = p_ref[...] * rdeg
    acc = jnp.dot(h, ws_ref[...], preferred_element_type=jnp.float32)
    acc += jnp.dot(hn, wn_ref[...], preferred_element_type=jnp.float32)
    res = jnp.maximum(acc + b_ref[...], 0.0)
    olo_ref[...] = res[:, :_HH]
    ohi_ref[...] = res[:, _HH:]
    ord_ref[...] = jnp.broadcast_to(rdeg, (rdeg.shape[0], 16))


def _make_dense(n, n_pad, blk=1024):
    grid = (n_pad // blk,)
    return pl.pallas_call(
        _dense_body,
        grid=grid,
        in_specs=[
            pl.BlockSpec((blk, _HH), lambda i: (i, 0)),
            pl.BlockSpec((blk, _HH), lambda i: (i, 0)),
            pl.BlockSpec((blk, 128), lambda i: (i, 0)),
            pl.BlockSpec((blk, 16), lambda i: (i, 0)),
            pl.BlockSpec((blk, 16), lambda i: (i, 0)),
            pl.BlockSpec((128, 128), lambda i: (0, 0)),
            pl.BlockSpec((128, 128), lambda i: (0, 0)),
            pl.BlockSpec((1, 128), lambda i: (0, 0)),
        ],
        out_specs=[pl.BlockSpec((blk, _HH), lambda i: (i, 0)),
                   pl.BlockSpec((blk, _HH), lambda i: (i, 0)),
                   pl.BlockSpec((blk, 16), lambda i: (i, 0))],
        out_shape=[jax.ShapeDtypeStruct((n, _HH), jnp.float32),
                   jax.ShapeDtypeStruct((n, _HH), jnp.float32),
                   jax.ShapeDtypeStruct((n, 16), jnp.float32)],
    )


def _head_body(gh_ref, gp_ref, gr_ref, ws1, wn1, b1_ref,
               w1a, w1b, w1c, bl1, gam, bet, w2, b2,
               hp_ref, hs_ref, ht_ref):
    b = hs_ref.shape[0]

    def mm(x, w):
        return jnp.dot(x, w[...], preferred_element_type=jnp.float32)

    rdeg = gr_ref[:, 0:1]
    h2 = mm(gh_ref[...], ws1) + mm(gp_ref[...] * rdeg, wn1) + b1_ref[...]
    h2 = jnp.maximum(h2, 0.0)
    ha, hb = h2[:b], h2[b:]
    z = mm(ha, w1a) + mm(hb, w1b) + mm(jnp.abs(ha - hb), w1c) + bl1[...]
    mu = jnp.mean(z, axis=0, keepdims=True)
    var = jnp.mean((z - mu) ** 2, axis=0, keepdims=True)
    zn = gam[...] * (z - mu) / jnp.sqrt(var + 1e-5) + bet[...]
    res = jnp.maximum(zn, 0.0)
    i = pl.program_id(0)

    @pl.when(i == 0)
    def _():
        hs_ref[...] = res
        hp_ref[...] = mm(res, w2) + b2[...]

    @pl.when(i == 1)
    def _():
        ht_ref[...] = res


def _make_head(b):
    w = lambda shape: pl.BlockSpec(shape, lambda i: tuple(0 for _ in shape))
    return pl.pallas_call(
        _head_body,
        grid=(2,),
        in_specs=[
            pl.BlockSpec((2 * b, 128), lambda i: (i, 0)),
            pl.BlockSpec((2 * b, 128), lambda i: (i, 0)),
            pl.BlockSpec((2 * b, 16), lambda i: (i, 0)),
            w((128, 128)), w((128, 128)), w((1, 128)),
            w((128, 128)), w((128, 128)), w((128, 128)),
            w((1, 128)), w((1, 128)), w((1, 128)),
            w((128, 128)), w((1, 128)),
        ],
        out_specs=[w((b, 128)), w((b, 128)), w((b, 128))],
        out_shape=[
            jax.ShapeDtypeStruct((b, 128), jnp.float32),
            jax.ShapeDtypeStruct((b, 128), jnp.float32),
            jax.ShapeDtypeStruct((b, 128), jnp.float32),
        ],
    )


def kernel(h, edge_index, x1, x2, x1_tar, x2_tar,
           W_self0, W_neigh0, b0, W_self1, W_neigh1, b1,
           W_lin1, b_lin1, bn_gamma, bn_beta, W_lin2, b_lin2):
    n, d = h.shape
    e = edge_index.shape[1]
    bsz = x1.shape[0]
    hdim = W_self0.shape[1]
    c_out = W_lin2.shape[1]
    n_pad = ((n + 1023) // 1024) * 1024
    chunk = 200
    b4 = 4 * bsz
    assert d == 128 and hdim == 128

    ei = edge_index.astype(jnp.int32)
    h_lo = h[:, :_HH]
    h_hi = h[:, _HH:]

    z64 = jnp.zeros((n_pad, _HH), jnp.float32)
    z16 = jnp.zeros((n_pad, 16), jnp.float32)
    ones16 = jnp.ones((chunk, 16), jnp.float32)

    seg0 = _make_seg(n, n_pad, e, chunk, b4, with_deg=True)
    seg1 = _make_seg(n, n_pad, e, chunk, b4, with_deg=False)
    dense = _make_dense(n, n_pad)
    head = _make_head(bsz)

    idx_all = jnp.concatenate([x1, x2, x1_tar, x2_tar]).astype(jnp.int32)

    p0, d0, d1 = seg0(h_lo, h_hi, ei, z64, z16, ones16)
    h1_lo, h1_hi, rd16 = dense(h_lo, h_hi, p0, d0, d1, W_self0, W_neigh0,
                               b0.reshape(1, 128))
    g_h, g_p, g_r = seg1(h1_lo, h1_hi, ei, z64, idx_all, rd16)

    w1a = W_lin1[0 * hdim:1 * hdim]
    w1b = W_lin1[1 * hdim:2 * hdim]
    w1c = W_lin1[2 * hdim:3 * hdim]
    w2p = jnp.zeros((hdim, 128), jnp.float32).at[:, :c_out].set(W_lin2)
    b2p = jnp.zeros((1, 128), jnp.float32).at[0, :c_out].set(b_lin2)

    hp_pad, hs, ht = head(g_h, g_p, g_r, W_self1, W_neigh1,
                          b1.reshape(1, 128), w1a, w1b, w1c,
                          b_lin1.reshape(1, 128), bn_gamma.reshape(1, 128),
                          bn_beta.reshape(1, 128), w2p, b2p)
    return hp_pad[:, :c_out], hs, ht


# speedup vs baseline: 12.8537x; 1.0032x over previous
import jax
import jax.numpy as jnp
from jax import lax
from jax.experimental import pallas as pl
from jax.experimental.pallas import tpu as pltpu
from jax.experimental.pallas import tpu_sc as plsc

_NC = 2
_NS = 16
_HH = 64


def _sc_mesh():
    return plsc.VectorSubcoreMesh(
        core_axis_name="c", subcore_axis_name="s",
        num_cores=_NC, num_subcores=_NS)


def _chunks(total, step):
    out, o = [], 0
    while o < total:
        sz = min(step, total - o)
        out.append((o, sz))
        o += sz
    assert all(sz % 8 == 0 for _, sz in out)
    return out


def _make_seg(n, n_pad, e, chunk, b4, with_deg):
    assert e % (_NS * chunk) == 0 and chunk % 8 == 0
    steps = e // (_NS * chunk)
    nb = 4
    assert steps % nb == 0
    half = steps // 2
    epw = e // _NS
    rpt = n_pad // _NS

    if with_deg:
        out_type = [jax.ShapeDtypeStruct((n_pad, 128), jnp.float32),
                    jax.ShapeDtypeStruct((n_pad, 16), jnp.float32),
                    jax.ShapeDtypeStruct((n_pad, 16), jnp.float32)]
    else:
        out_type = [jax.ShapeDtypeStruct((b4, 128), jnp.float32),
                    jax.ShapeDtypeStruct((b4, 128), jnp.float32),
                    jax.ShapeDtypeStruct((b4, 16), jnp.float32)]
    scratch = [
        *[pltpu.VMEM((2, chunk), jnp.int32) for _ in range(nb)],
        *[pltpu.VMEM((chunk, _HH), jnp.float32) for _ in range(nb)],
        pltpu.VMEM_SHARED((n_pad, _HH), jnp.float32),
        pltpu.SemaphoreType.DMA,
        pltpu.SemaphoreType.DMA,
    ]
    if with_deg:
        scratch += [
            pltpu.VMEM((chunk, 16), jnp.float32),
            pltpu.VMEM_SHARED((n_pad, 16), jnp.float32),
        ]
    else:
        scratch += [
            pltpu.VMEM((chunk, 16), jnp.float32),
            pltpu.VMEM((chunk, 16), jnp.float32),
        ]

    def body(*refs):
        if with_deg:
            (h_lo, h_hi, ei, z64, z16, ones_h, out, d0, d1) = refs[:9]
            rest = refs[9:]
        else:
            (h_lo, h_hi, ei, z64, idx_hbm, rd,
             g_h, g_p, g_r) = refs[:9]
            rest = refs[9:]
        idx = rest[0:nb]
        rows = rest[nb:2 * nb]
        acc, gsem, isem = rest[2 * nb:2 * nb + 3]
        if with_deg:
            ones_v, dacc = rest[2 * nb + 3:]
        else:
            db0, db1 = rest[2 * nb + 3:]
        c = lax.axis_index("c")
        s = lax.axis_index("s")
        r0 = s * rpt
        pltpu.sync_copy(z64.at[pl.ds(r0, rpt)], acc.at[pl.ds(r0, rpt)])
        if with_deg:
            pltpu.sync_copy(z16.at[pl.ds(r0, rpt)], dacc.at[pl.ds(r0, rpt)])
            pltpu.sync_copy(ones_h, ones_v)
        plsc.subcore_barrier()
        base = s * epw

        def off(g):
            gc = jnp.minimum(g, steps - 1) if isinstance(g, jax.Array) \
                else min(g, steps - 1)
            return pl.multiple_of(base + gc * chunk, 8)

        def idx_start(g, slot):
            pltpu.async_copy(ei.at[:, pl.ds(off(g), chunk)], idx[slot], isem)

        def idx_wait(slot):
            pltpu.make_async_copy(ei.at[:, pl.ds(base, chunk)], idx[slot],
                                  isem).wait()

        def gather_start(slot):
            @pl.when(c == 0)
            def _():
                pltpu.async_copy(h_lo.at[idx[slot].at[0]], rows[slot], gsem)

            @pl.when(c == 1)
            def _():
                pltpu.async_copy(h_hi.at[idx[slot].at[0]], rows[slot], gsem)

        def gather_wait(slot):
            pltpu.make_async_copy(h_lo.at[idx[slot].at[0]], rows[slot],
                                  gsem).wait()

        pltpu.sync_copy(ei.at[:, pl.ds(off(0), chunk)], idx[0])
        pltpu.sync_copy(ei.at[:, pl.ds(off(1), chunk)], idx[1])
        gather_start(0)
        gather_start(1)
        idx_start(2, 2)

        def phase(g, b):
            b2 = (b + 2) % nb
            b3 = (b + 3) % nb
            idx_wait(b2)
            gather_start(b2)
            idx_start(g + 3, b3)
            gather_wait(b)
            pltpu.sync_copy(rows[b], acc.at[idx[b].at[1]], add=True)
            if with_deg:
                @pl.when((g < half) == (c == 0))
                def _():
                    pltpu.sync_copy(ones_v, dacc.at[idx[b].at[1]], add=True)

        def outer(k, carry):
            g0 = k * nb
            for b in range(nb):
                phase(g0 + b, b)
            return carry

        lax.fori_loop(0, steps // nb, outer, 0)
        gather_wait(0)
        gather_wait(1)
        idx_wait(2)
        plsc.subcore_barrier()

        if with_deg:
            pltpu.sync_copy(acc.at[pl.ds(r0, rpt)],
                            out.at[pl.ds(r0, rpt), pl.ds(c * _HH, _HH)])

            @pl.when(c == 0)
            def _():
                pltpu.sync_copy(dacc.at[pl.ds(r0, rpt)], d0.at[pl.ds(r0, rpt)])

            @pl.when(c == 1)
            def _():
                pltpu.sync_copy(dacc.at[pl.ds(r0, rpt)], d1.at[pl.ds(r0, rpt)])
            return

        wid = c * _NS + s
        bpw = b4 // (_NC * _NS)
        gb = wid * bpw
        achunks = _chunks(bpw, chunk)
        asets = [(rows[0], rows[1], db0, idx[0]),
                 (rows[2], rows[3], db1, idx[2])]

        def a_fire(k):
            o, sz = achunks[k]
            r1, r2, dbuf, ix = asets[k % 2]
            pltpu.sync_copy(idx_hbm.at[pl.ds(gb + o, sz)],
                            ix.at[0, pl.ds(0, sz)])
            ib = ix.at[0, pl.ds(0, sz)]
            pltpu.async_copy(h_lo.at[ib], r1.at[pl.ds(0, sz)], gsem)
            pltpu.async_copy(h_hi.at[ib], r2.at[pl.ds(0, sz)], gsem)
            pltpu.async_copy(rd.at[ib], dbuf.at[pl.ds(0, sz)], gsem)

        def a_drain(k):
            o, sz = achunks[k]
            r1, r2, dbuf, ix = asets[k % 2]
            ib = ix.at[0, pl.ds(0, sz)]
            pltpu.make_async_copy(h_lo.at[ib], r1.at[pl.ds(0, sz)],
                                  gsem).wait()
            pltpu.make_async_copy(h_hi.at[ib], r2.at[pl.ds(0, sz)],
                                  gsem).wait()
            pltpu.make_async_copy(rd.at[ib], dbuf.at[pl.ds(0, sz)],
                                  gsem).wait()
            pltpu.sync_copy(r1.at[pl.ds(0, sz)],
                            g_h.at[pl.ds(gb + o, sz), pl.ds(0, _HH)])
            pltpu.sync_copy(r2.at[pl.ds(0, sz)],
                            g_h.at[pl.ds(gb + o, sz), pl.ds(_HH, _HH)])
            pltpu.sync_copy(dbuf.at[pl.ds(0, sz)], g_r.at[pl.ds(gb + o, sz)])

        a_fire(0)
        for k in range(len(achunks)):
            if k + 1 < len(achunks):
                a_fire(k + 1)
            a_drain(k)

        ppw = b4 // _NS
        pb = s * ppw
        bchunks = _chunks(ppw, chunk)
        bsets = [(rows[0], idx[1]), (rows[1], idx[3])]

        def b_fire(k):
            o, sz = bchunks[k]
            rb, ix = bsets[k % 2]
            pltpu.sync_copy(idx_hbm.at[pl.ds(pb + o, sz)],
                            ix.at[0, pl.ds(0, sz)])
            pltpu.async_copy(acc.at[ix.at[0, pl.ds(0, sz)]],
                             rb.at[pl.ds(0, sz)], gsem)

        def b_drain(k):
            o, sz = bchunks[k]
            rb, ix = bsets[k % 2]
            pltpu.make_async_copy(acc.at[ix.at[0, pl.ds(0, sz)]],
                                  rb.at[pl.ds(0, sz)], gsem).wait()
            pltpu.sync_copy(rb.at[pl.ds(0, sz)],
                            g_p.at[pl.ds(pb + o, sz), pl.ds(c * _HH, _HH)])

        b_fire(0)
        for k in range(len(bchunks)):
            if k + 1 < len(bchunks):
                b_fire(k + 1)
            b_drain(k)

    return pl.kernel(body, out_type=tuple(out_type), mesh=_sc_mesh(),
                     scratch_types=tuple(scratch),
                     compiler_params=pltpu.CompilerParams(
                         use_tc_tiling_on_sc=False))


def _dense_body(h_ref, p_ref, d0_ref, d1_ref, ws_ref,
                wn_ref, b_ref, olo_ref, ohi_ref, ord_ref):
    h = h_ref[...]
    rdeg = 1.0 / jnp.maximum(d0_ref[:, 0:1] + d1_ref[:, 0:1], 1.0)
    hn = p_ref[...] * rdeg
    acc = jnp.dot(h, ws_ref[...], preferred_element_type=jnp.float32)
    acc += jnp.dot(hn, wn_ref[...], preferred_element_type=jnp.float32)
    res = jnp.maximum(acc + b_ref[...], 0.0)
    olo_ref[...] = res[:, :_HH]
    ohi_ref[...] = res[:, _HH:]
    ord_ref[...] = jnp.broadcast_to(rdeg, (rdeg.shape[0], 16))


def _make_dense(n, n_pad, blk=1024):
    grid = (n_pad // blk,)
    return pl.pallas_call(
        _dense_body,
        grid=grid,
        in_specs=[
            pl.BlockSpec((blk, 128), lambda i: (i, 0)),
            pl.BlockSpec((blk, 128), lambda i: (i, 0)),
            pl.BlockSpec((blk, 16), lambda i: (i, 0)),
            pl.BlockSpec((blk, 16), lambda i: (i, 0)),
            pl.BlockSpec((128, 128), lambda i: (0, 0)),
            pl.BlockSpec((128, 128), lambda i: (0, 0)),
            pl.BlockSpec((1, 128), lambda i: (0, 0)),
        ],
        out_specs=[pl.BlockSpec((blk, _HH), lambda i: (i, 0)),
                   pl.BlockSpec((blk, _HH), lambda i: (i, 0)),
                   pl.BlockSpec((blk, 16), lambda i: (i, 0))],
        out_shape=[jax.ShapeDtypeStruct((n, _HH), jnp.float32),
                   jax.ShapeDtypeStruct((n, _HH), jnp.float32),
                   jax.ShapeDtypeStruct((n, 16), jnp.float32)],
    )


def _head_body(gh_ref, gp_ref, gr_ref, ws1, wn1, b1_ref,
               w1a, w1b, w1c, bl1, gam, bet, w2, b2,
               hp_ref, hs_ref, ht_ref):
    b = hs_ref.shape[0]

    def mm(x, w):
        return jnp.dot(x, w[...], preferred_element_type=jnp.float32)

    rdeg = gr_ref[:, 0:1]
    h2 = mm(gh_ref[...], ws1) + mm(gp_ref[...] * rdeg, wn1) + b1_ref[...]
    h2 = jnp.maximum(h2, 0.0)
    ha, hb = h2[:b], h2[b:]
    z = mm(ha, w1a) + mm(hb, w1b) + mm(jnp.abs(ha - hb), w1c) + bl1[...]
    mu = jnp.mean(z, axis=0, keepdims=True)
    var = jnp.mean((z - mu) ** 2, axis=0, keepdims=True)
    zn = gam[...] * (z - mu) / jnp.sqrt(var + 1e-5) + bet[...]
    res = jnp.maximum(zn, 0.0)
    i = pl.program_id(0)

    @pl.when(i == 0)
    def _():
        hs_ref[...] = res
        hp_ref[...] = mm(res, w2) + b2[...]

    @pl.when(i == 1)
    def _():
        ht_ref[...] = res


def _make_head(b):
    w = lambda shape: pl.BlockSpec(shape, lambda i: tuple(0 for _ in shape))
    return pl.pallas_call(
        _head_body,
        grid=(2,),
        in_specs=[
            pl.BlockSpec((2 * b, 128), lambda i: (i, 0)),
            pl.BlockSpec((2 * b, 128), lambda i: (i, 0)),
            pl.BlockSpec((2 * b, 16), lambda i: (i, 0)),
            w((128, 128)), w((128, 128)), w((1, 128)),
            w((128, 128)), w((128, 128)), w((128, 128)),
            w((1, 128)), w((1, 128)), w((1, 128)),
            w((128, 128)), w((1, 128)),
        ],
        out_specs=[w((b, 128)), w((b, 128)), w((b, 128))],
        out_shape=[
            jax.ShapeDtypeStruct((b, 128), jnp.float32),
            jax.ShapeDtypeStruct((b, 128), jnp.float32),
            jax.ShapeDtypeStruct((b, 128), jnp.float32),
        ],
    )


def kernel(h, edge_index, x1, x2, x1_tar, x2_tar,
           W_self0, W_neigh0, b0, W_self1, W_neigh1, b1,
           W_lin1, b_lin1, bn_gamma, bn_beta, W_lin2, b_lin2):
    n, d = h.shape
    e = edge_index.shape[1]
    bsz = x1.shape[0]
    hdim = W_self0.shape[1]
    c_out = W_lin2.shape[1]
    n_pad = ((n + 1023) // 1024) * 1024
    chunk = 200
    b4 = 4 * bsz
    assert d == 128 and hdim == 128

    ei = edge_index.astype(jnp.int32)
    h_lo = h[:, :_HH]
    h_hi = h[:, _HH:]

    z64 = jnp.zeros((n_pad, _HH), jnp.float32)
    z16 = jnp.zeros((n_pad, 16), jnp.float32)
    ones16 = jnp.ones((chunk, 16), jnp.float32)

    seg0 = _make_seg(n, n_pad, e, chunk, b4, with_deg=True)
    seg1 = _make_seg(n, n_pad, e, chunk, b4, with_deg=False)
    dense = _make_dense(n, n_pad)
    head = _make_head(bsz)

    idx_all = jnp.concatenate([x1, x2, x1_tar, x2_tar]).astype(jnp.int32)

    p0, d0, d1 = seg0(h_lo, h_hi, ei, z64, z16, ones16)
    h1_lo, h1_hi, rd16 = dense(h, p0, d0, d1, W_self0, W_neigh0,
                               b0.reshape(1, 128))
    g_h, g_p, g_r = seg1(h1_lo, h1_hi, ei, z64, idx_all, rd16)

    w1a = W_lin1[0 * hdim:1 * hdim]
    w1b = W_lin1[1 * hdim:2 * hdim]
    w1c = W_lin1[2 * hdim:3 * hdim]
    w2p = jnp.zeros((hdim, 128), jnp.float32).at[:, :c_out].set(W_lin2)
    b2p = jnp.zeros((1, 128), jnp.float32).at[0, :c_out].set(b_lin2)

    hp_pad, hs, ht = head(g_h, g_p, g_r, W_self1, W_neigh1,
                          b1.reshape(1, 128), w1a, w1b, w1c,
                          b_lin1.reshape(1, 128), bn_gamma.reshape(1, 128),
                          bn_beta.reshape(1, 128), w2p, b2p)
    return hp_pad[:, :c_out], hs, ht
